# baseline (speedup 1.0000x reference)
import functools

import jax
import jax.numpy as jnp
from jax import lax
from jax.experimental import pallas as pl
from jax.experimental.pallas import tpu as pltpu
from jax.experimental.pallas import tpu_sc as plsc

NC = 2
NS = 16
NW = NC * NS
L = 16

VOCAB = 100000
HID = 128
B = 1024
S = 200
HC = HID // L
P = 4
NTOK = B * S
TPP = NTOK // P
TPW = TPP // NW
K = 80
CHP = TPW // K
NB = 2
UNROLL = 8
SB = 256
EPS = 1e-12

_MESH = plsc.VectorSubcoreMesh(
    core_axis_name="c", subcore_axis_name="s", num_cores=NC, num_subcores=NS
)


@functools.partial(
    pl.kernel,
    mesh=_MESH,
    compiler_params=pltpu.CompilerParams(needs_layout_passes=False),
    out_type=jax.ShapeDtypeStruct((TPP, HID), jnp.float32),
    scratch_types=(
        [pltpu.VMEM((CHP, K), jnp.int32)]
        + [pltpu.VMEM((K, HID), jnp.float32)] * NB
        + [pltpu.VMEM((K, HID), jnp.float32)] * NB
        + [
            pltpu.VMEM((SB, HID), jnp.float32),
            pltpu.VMEM((HID,), jnp.float32),
        ]
        + [pltpu.SemaphoreType.DMA] * (2 * NB)
    ),
)
def _sc_gather(ids_hbm, pe_hbm, tte_hbm, table_hbm, out_hbm, idx_v, *rest):
    rows = rest[0:NB]
    outb = rest[NB:2 * NB]
    bias_v, tte_v = rest[2 * NB:2 * NB + 2]
    gsem = rest[2 * NB + 2:3 * NB + 2]
    ssem = rest[3 * NB + 2:4 * NB + 2]
    wid = lax.axis_index("s") * NC + lax.axis_index("c")

    pltpu.sync_copy(pe_hbm, bias_v.at[pl.ds(0, S)])
    pltpu.sync_copy(pe_hbm.at[pl.ds(0, SB - S)], bias_v.at[pl.ds(S, SB - S)])
    pltpu.sync_copy(tte_hbm, tte_v)
    pltpu.sync_copy(ids_hbm.at[wid], idx_v)

    ttec = [tte_v[pl.ds(c * L, L)] for c in range(HC)]

    def bias_body(t, carry):
        for c in range(HC):
            bias_v[t, pl.ds(c * L, L)] = bias_v[t, pl.ds(c * L, L)] + ttec[c]
        return carry

    lax.fori_loop(0, SB, bias_body, 0)

    base = wid * TPW

    for b in range(NB):
        pltpu.async_copy(table_hbm.at[idx_v.at[b]], rows[b], gsem[b])

    def ring_body(jj, carry):
        for b in range(NB):
            j = jj * NB + b
            rows_b, outb_b, gsem_b, ssem_b = rows[b], outb[b], gsem[b], ssem[b]
            pltpu.make_async_copy(
                table_hbm.at[pl.ds(0, K)], rows_b, gsem_b).wait()

            @pl.when(jj > 0)
            def _():
                pltpu.make_async_copy(
                    outb_b, out_hbm.at[pl.ds(0, K)], ssem_b).wait()

            poff = lax.rem(j * K, S)

            @plsc.parallel_loop(0, K, 1, unroll=UNROLL)
            def _(t):
                for c in range(HC):
                    outb_b[t, pl.ds(c * L, L)] = (
                        rows_b[t, pl.ds(c * L, L)]
                        + bias_v[poff + t, pl.ds(c * L, L)])

            @pl.when(j + NB < CHP)
            def _():
                pltpu.async_copy(
                    table_hbm.at[idx_v.at[j + NB]], rows_b, gsem_b)

            pltpu.async_copy(
                outb_b, out_hbm.at[pl.ds(base + j * K, K)], ssem_b)
        return carry

    lax.fori_loop(0, CHP // NB, ring_body, 0)

    for b in range(NB):
        pltpu.make_async_copy(
            outb[b], out_hbm.at[pl.ds(0, K)], ssem[b]).wait()


_BLK = 2048


def _tc_ln_body(x_ref, gam_ref, bet_ref, o_ref):
    x = x_ref[...]
    m = jnp.mean(x, axis=1, keepdims=True)
    xc = x - m
    v = jnp.mean(xc * xc, axis=1, keepdims=True)
    r = lax.rsqrt(v + EPS)
    o_ref[...] = xc * r * gam_ref[...] + bet_ref[...]


_tc_ln = pl.pallas_call(
    _tc_ln_body,
    grid=(TPP // _BLK,),
    in_specs=[
        pl.BlockSpec((_BLK, HID), lambda g: (g, 0)),
        pl.BlockSpec((1, HID), lambda g: (0, 0)),
        pl.BlockSpec((1, HID), lambda g: (0, 0)),
    ],
    out_specs=pl.BlockSpec((_BLK, HID), lambda g: (g, 0)),
    out_shape=jax.ShapeDtypeStruct((TPP, HID), jnp.float32),
)


def kernel(input_ids, word_embeddings, position_embeddings,
           token_type_embeddings, ln_gamma, ln_beta):
    b, s = input_ids.shape
    _, h = word_embeddings.shape
    ids4 = input_ids.reshape(P, NW, CHP, K)
    pe = position_embeddings[:s]
    tte0 = token_type_embeddings[0]
    g2 = ln_gamma.reshape(1, h)
    b2 = ln_beta.reshape(1, h)
    outs = []
    for p in range(P):
        tmp = _sc_gather(ids4[p], pe, tte0, word_embeddings)
        outs.append(_tc_ln(tmp, g2, b2))
    return jnp.concatenate(outs, axis=0).reshape(b, s, h)
